# baseline (speedup 1.0000x reference)
import functools

import jax
import jax.numpy as jnp
from jax import lax
from jax.experimental import pallas as pl
from jax.experimental.pallas import tpu as pltpu
from jax.experimental.pallas import tpu_sc as plsc

_L = 16
_NW = 32
_CH = 6400
_C = 6
_V = 20


def _sc_lookup(n_total):
    chunk = n_total // _NW
    steps = chunk // _CH
    assert steps % 2 == 0 and steps >= 4
    mesh = plsc.VectorSubcoreMesh(core_axis_name="c", subcore_axis_name="s")

    @functools.partial(
        pl.kernel,
        mesh=mesh,
        compiler_params=pltpu.CompilerParams(needs_layout_passes=False),
        out_type=jax.ShapeDtypeStruct((1, _C, n_total), jnp.float32),
        scratch_types=[
            pltpu.VMEM((2, _CH), jnp.int32),
            pltpu.VMEM((2, _C, _CH), jnp.float32),
            pltpu.VMEM((128,), jnp.float32),
            pltpu.SemaphoreType.DMA,
            pltpu.SemaphoreType.DMA,
            pltpu.SemaphoreType.DMA,
            pltpu.SemaphoreType.DMA,
        ],
    )
    def k(idx_hbm, tab_hbm, out_hbm, idx_v, out_v, tab_v,
          in0, in1, o0, o1):
        in_sems = (in0, in1)
        out_sems = (o0, o1)
        wid = lax.axis_index("s") * 2 + lax.axis_index("c")
        base0 = wid * chunk
        pltpu.sync_copy(tab_hbm, tab_v)

        def issue_in(s, b):
            sw = lax.rem(s, steps)
            pltpu.async_copy(
                idx_hbm.at[pl.ds(base0 + sw * _CH, _CH)], idx_v.at[b],
                in_sems[b])

        def wait_in(b):
            pltpu.make_async_copy(
                idx_hbm.at[pl.ds(0, _CH)], idx_v.at[b], in_sems[b]).wait()

        def issue_out(s, b):
            pltpu.async_copy(
                out_v.at[b], out_hbm.at[0, :, pl.ds(base0 + s * _CH, _CH)],
                out_sems[b])

        def wait_out(b):
            pltpu.make_async_copy(
                out_v.at[b], out_hbm.at[0, :, pl.ds(0, _CH)],
                out_sems[b]).wait()

        def compute(b):
            @plsc.parallel_loop(0, _CH, _L, unroll=8)
            def inner(e):
                idx_vec = idx_v[b, pl.ds(e, _L)]
                for c in range(_C):
                    off = idx_vec + (c * _V) if c else idx_vec
                    out_v[b, c, pl.ds(e, _L)] = plsc.load_gather(
                        tab_v, [off])

        issue_in(0, 0)
        issue_in(1, 1)

        for b in range(2):
            wait_in(b)
            compute(b)
            issue_in(2 + b, b)
            issue_out(b, b)

        def pair(t, carry):
            s = t * 2
            for b in range(2):
                wait_in(b)
                wait_out(b)
                compute(b)
                issue_in(s + 2 + b, b)
                issue_out(s + b, b)
            return carry

        lax.fori_loop(1, steps // 2, pair, 0)

        for b in range(2):
            wait_in(b)
            wait_out(b)

    return k


def kernel(primary, table):
    n = primary.shape[0] * primary.shape[1]
    idx = primary.reshape(-1).astype(jnp.int32)
    tab_t = jnp.zeros((128,), jnp.float32).at[: _C * _V].set(
        table.T.reshape(-1))
    return _sc_lookup(n)(idx, tab_t)


# speedup vs baseline: 1.0023x; 1.0023x over previous
import functools

import jax
import jax.numpy as jnp
from jax import lax
from jax.experimental import pallas as pl
from jax.experimental.pallas import tpu as pltpu
from jax.experimental.pallas import tpu_sc as plsc

_L = 16
_NW = 32
_CH = 6400
_C = 6
_V = 20


def _sc_lookup(n_total):
    chunk = n_total // _NW
    steps = chunk // _CH
    assert steps % 2 == 0 and steps >= 4
    mesh = plsc.VectorSubcoreMesh(core_axis_name="c", subcore_axis_name="s")

    @functools.partial(
        pl.kernel,
        mesh=mesh,
        compiler_params=pltpu.CompilerParams(
            needs_layout_passes=False, use_tc_tiling_on_sc=True),
        out_type=jax.ShapeDtypeStruct((1, _C, n_total), jnp.float32),
        scratch_types=[
            pltpu.VMEM((2, _CH), jnp.int32),
            pltpu.VMEM((2, _C, _CH), jnp.float32),
            pltpu.VMEM((128,), jnp.float32),
            pltpu.SemaphoreType.DMA,
            pltpu.SemaphoreType.DMA,
            pltpu.SemaphoreType.DMA,
            pltpu.SemaphoreType.DMA,
        ],
    )
    def k(idx_hbm, tab_hbm, out_hbm, idx_v, out_v, tab_v,
          in0, in1, o0, o1):
        in_sems = (in0, in1)
        out_sems = (o0, o1)
        wid = lax.axis_index("s") * 2 + lax.axis_index("c")
        base0 = wid * chunk
        pltpu.sync_copy(tab_hbm, tab_v)

        def issue_in(s, b):
            sw = lax.rem(s, steps)
            pltpu.async_copy(
                idx_hbm.at[pl.ds(base0 + sw * _CH, _CH)], idx_v.at[b],
                in_sems[b])

        def wait_in(b):
            pltpu.make_async_copy(
                idx_hbm.at[pl.ds(0, _CH)], idx_v.at[b], in_sems[b]).wait()

        def issue_out(s, b):
            pltpu.async_copy(
                out_v.at[b], out_hbm.at[0, :, pl.ds(base0 + s * _CH, _CH)],
                out_sems[b])

        def wait_out(b):
            pltpu.make_async_copy(
                out_v.at[b], out_hbm.at[0, :, pl.ds(0, _CH)],
                out_sems[b]).wait()

        def compute(b):
            @plsc.parallel_loop(0, _CH, _L, unroll=8)
            def inner(e):
                idx_vec = idx_v[b, pl.ds(e, _L)]
                for c in range(_C):
                    off = idx_vec + (c * _V) if c else idx_vec
                    out_v[b, c, pl.ds(e, _L)] = plsc.load_gather(
                        tab_v, [off])

        issue_in(0, 0)
        issue_in(1, 1)

        for b in range(2):
            wait_in(b)
            compute(b)
            issue_in(2 + b, b)
            issue_out(b, b)

        def pair(t, carry):
            s = t * 2
            for b in range(2):
                wait_in(b)
                wait_out(b)
                compute(b)
                issue_in(s + 2 + b, b)
                issue_out(s + b, b)
            return carry

        lax.fori_loop(1, steps // 2, pair, 0)

        for b in range(2):
            wait_in(b)
            wait_out(b)

    return k


def kernel(primary, table):
    n = primary.shape[0] * primary.shape[1]
    idx = primary.reshape(-1).astype(jnp.int32)
    tab_t = jnp.zeros((128,), jnp.float32).at[: _C * _V].set(
        table.T.reshape(-1))
    return _sc_lookup(n)(idx, tab_t)


# speedup vs baseline: 1.1023x; 1.0998x over previous
import functools

import jax
import jax.numpy as jnp
from jax import lax
from jax.experimental import pallas as pl
from jax.experimental.pallas import tpu as pltpu
from jax.experimental.pallas import tpu_sc as plsc

_L = 16
_NW = 32
_CH = 6400
_C = 6
_V = 20


def _sc_lookup(n_total):
    chunk = n_total // _NW
    steps = chunk // _CH
    assert steps % 2 == 0 and steps >= 4
    mesh = plsc.VectorSubcoreMesh(core_axis_name="c", subcore_axis_name="s")

    @functools.partial(
        pl.kernel,
        mesh=mesh,
        compiler_params=pltpu.CompilerParams(
            needs_layout_passes=False, use_tc_tiling_on_sc=True),
        out_type=jax.ShapeDtypeStruct((_C, n_total), jnp.float32),
        scratch_types=[
            pltpu.VMEM((2, _CH), jnp.int32),
            pltpu.VMEM((2, _C, _CH), jnp.float32),
            pltpu.VMEM((128,), jnp.float32),
            pltpu.SemaphoreType.DMA,
            pltpu.SemaphoreType.DMA,
            pltpu.SemaphoreType.DMA,
            pltpu.SemaphoreType.DMA,
        ],
    )
    def k(idx_hbm, tab_hbm, out_hbm, idx_v, out_v, tab_v,
          in0, in1, o0, o1):
        in_sems = (in0, in1)
        out_sems = (o0, o1)
        wid = lax.axis_index("s") * 2 + lax.axis_index("c")
        base0 = wid * chunk
        pltpu.sync_copy(tab_hbm, tab_v)

        def issue_in(s, b):
            sw = lax.rem(s, steps)
            pltpu.async_copy(
                idx_hbm.at[pl.ds(base0 + sw * _CH, _CH)], idx_v.at[b],
                in_sems[b])

        def wait_in(b):
            pltpu.make_async_copy(
                idx_hbm.at[pl.ds(0, _CH)], idx_v.at[b], in_sems[b]).wait()

        def issue_out(s, b):
            pltpu.async_copy(
                out_v.at[b], out_hbm.at[:, pl.ds(base0 + s * _CH, _CH)],
                out_sems[b])

        def wait_out(b):
            pltpu.make_async_copy(
                out_v.at[b], out_hbm.at[:, pl.ds(0, _CH)],
                out_sems[b]).wait()

        def compute(b):
            @plsc.parallel_loop(0, _CH, _L, unroll=8)
            def inner(e):
                idx_vec = idx_v[b, pl.ds(e, _L)]
                for c in range(_C):
                    off = idx_vec + (c * _V) if c else idx_vec
                    out_v[b, c, pl.ds(e, _L)] = plsc.load_gather(
                        tab_v, [off])

        issue_in(0, 0)
        issue_in(1, 1)

        for b in range(2):
            wait_in(b)
            compute(b)
            issue_in(2 + b, b)
            issue_out(b, b)

        def pair(t, carry):
            s = t * 2
            for b in range(2):
                wait_in(b)
                wait_out(b)
                compute(b)
                issue_in(s + 2 + b, b)
                issue_out(s + b, b)
            return carry

        lax.fori_loop(1, steps // 2, pair, 0)

        for b in range(2):
            wait_in(b)
            wait_out(b)

    return k


def kernel(primary, table):
    n = primary.shape[0] * primary.shape[1]
    idx = primary.reshape(-1).astype(jnp.int32)
    tab_t = jnp.zeros((128,), jnp.float32).at[: _C * _V].set(
        table.T.reshape(-1))
    return _sc_lookup(n)(idx, tab_t).reshape(1, _C, n)


# speedup vs baseline: 1.1900x; 1.0796x over previous
import functools

import jax
import jax.numpy as jnp
from jax import lax
from jax.experimental import pallas as pl
from jax.experimental.pallas import tpu as pltpu
from jax.experimental.pallas import tpu_sc as plsc

_L = 16
_NW = 32
_RS = 32
_C = 6
_V = 20


def _sc_lookup(n_rows, n_cols):
    rows_w = n_rows // _NW
    steps = rows_w // _RS
    assert steps % 2 == 0 and steps >= 4
    ch = _RS * n_cols
    n_total = n_rows * n_cols
    full = n_cols // _L
    offs = [g * _L for g in range(full)]
    if n_cols % _L:
        offs.append(n_cols - _L)
    mesh = plsc.VectorSubcoreMesh(core_axis_name="c", subcore_axis_name="s")

    @functools.partial(
        pl.kernel,
        mesh=mesh,
        compiler_params=pltpu.CompilerParams(needs_layout_passes=False),
        out_type=jax.ShapeDtypeStruct((_C, n_total), jnp.float32),
        scratch_types=[
            pltpu.VMEM((2, _RS, n_cols), jnp.int32),
            pltpu.VMEM((2, _C, ch), jnp.float32),
            pltpu.VMEM((128,), jnp.float32),
            pltpu.SemaphoreType.DMA,
            pltpu.SemaphoreType.DMA,
            pltpu.SemaphoreType.DMA,
            pltpu.SemaphoreType.DMA,
        ],
    )
    def k(idx_hbm, tab_hbm, out_hbm, idx_v, out_v, tab_v,
          in0, in1, o0, o1):
        in_sems = (in0, in1)
        out_sems = (o0, o1)
        wid = lax.axis_index("s") * 2 + lax.axis_index("c")
        row0 = wid * rows_w
        pltpu.sync_copy(tab_hbm, tab_v)

        def issue_in(s, b):
            sw = lax.rem(s, steps)
            pltpu.async_copy(
                idx_hbm.at[pl.ds(row0 + sw * _RS, _RS), :], idx_v.at[b],
                in_sems[b])

        def wait_in(b):
            pltpu.make_async_copy(
                idx_hbm.at[pl.ds(0, _RS), :], idx_v.at[b], in_sems[b]).wait()

        def issue_out(s, b):
            base = (row0 + s * _RS) * n_cols
            pltpu.async_copy(
                out_v.at[b], out_hbm.at[:, pl.ds(base, ch)], out_sems[b])

        def wait_out(b):
            pltpu.make_async_copy(
                out_v.at[b], out_hbm.at[:, pl.ds(0, ch)], out_sems[b]).wait()

        def compute(b):
            @plsc.parallel_loop(0, _RS, 1)
            def row_body(r):
                col0 = r * n_cols
                for off in offs:
                    idx_vec = idx_v[b, r, pl.ds(off, _L)]
                    for c in range(_C):
                        o = idx_vec + (c * _V) if c else idx_vec
                        out_v[b, c, pl.ds(col0 + off, _L)] = (
                            plsc.load_gather(tab_v, [o]))

        issue_in(0, 0)
        issue_in(1, 1)

        for b in range(2):
            wait_in(b)
            compute(b)
            issue_in(2 + b, b)
            issue_out(b, b)

        def pair(t, carry):
            s = t * 2
            for b in range(2):
                wait_in(b)
                wait_out(b)
                compute(b)
                issue_in(s + 2 + b, b)
                issue_out(s + b, b)
            return carry

        lax.fori_loop(1, steps // 2, pair, 0)

        for b in range(2):
            wait_in(b)
            wait_out(b)

    return k


def kernel(primary, table):
    n_rows, n_cols = primary.shape
    tab_t = jnp.zeros((128,), jnp.float32).at[: _C * _V].set(
        table.T.reshape(-1))
    out = _sc_lookup(n_rows, n_cols)(primary.astype(jnp.int32), tab_t)
    return out.reshape(1, _C, n_rows * n_cols)


# speedup vs baseline: 1.2603x; 1.0591x over previous
import functools

import jax
import jax.numpy as jnp
from jax import lax
from jax.experimental import pallas as pl
from jax.experimental.pallas import tpu as pltpu
from jax.experimental.pallas import tpu_sc as plsc

_L = 16
_NW = 32
_RS = 32
_C = 6
_V = 20


def _sc_lookup(n_rows, n_cols):
    rows_w = n_rows // _NW
    steps = rows_w // _RS
    assert steps % 2 == 0 and steps >= 4
    ch = _RS * n_cols
    n_total = n_rows * n_cols
    pair_w = 2 * n_cols
    groups = pair_w // _L
    mesh = plsc.VectorSubcoreMesh(core_axis_name="c", subcore_axis_name="s")

    @functools.partial(
        pl.kernel,
        mesh=mesh,
        compiler_params=pltpu.CompilerParams(needs_layout_passes=False),
        out_type=jax.ShapeDtypeStruct((_C, n_total), jnp.float32),
        scratch_types=[
            pltpu.VMEM((2, _RS, n_cols), jnp.int32),
            pltpu.VMEM((2, _C, ch), jnp.float32),
            pltpu.VMEM((128,), jnp.float32),
            pltpu.SemaphoreType.DMA,
            pltpu.SemaphoreType.DMA,
            pltpu.SemaphoreType.DMA,
            pltpu.SemaphoreType.DMA,
        ],
    )
    def k(idx_hbm, tab_hbm, out_hbm, idx_v, out_v, tab_v,
          in0, in1, o0, o1):
        in_sems = (in0, in1)
        out_sems = (o0, o1)
        wid = lax.axis_index("s") * 2 + lax.axis_index("c")
        row0 = wid * rows_w
        pltpu.sync_copy(tab_hbm, tab_v)

        def issue_in(s, b):
            sw = lax.rem(s, steps)
            pltpu.async_copy(
                idx_hbm.at[pl.ds(row0 + sw * _RS, _RS), :], idx_v.at[b],
                in_sems[b])

        def wait_in(b):
            pltpu.make_async_copy(
                idx_hbm.at[pl.ds(0, _RS), :], idx_v.at[b], in_sems[b]).wait()

        def issue_out(s, b):
            base = (row0 + s * _RS) * n_cols
            pltpu.async_copy(
                out_v.at[b], out_hbm.at[:, pl.ds(base, ch)], out_sems[b])

        def wait_out(b):
            pltpu.make_async_copy(
                out_v.at[b], out_hbm.at[:, pl.ds(0, ch)], out_sems[b]).wait()

        iota = jnp.arange(_L, dtype=jnp.int32)

        def compute(b):
            bfull = iota * 0 + b

            @plsc.parallel_loop(0, ch // _L, 1, unroll=2)
            def g_body(i):
                e0 = i * _L
                e = iota + e0
                r = e // n_cols
                col = e - r * n_cols
                idx_vec = plsc.load_gather(idx_v, [bfull, r, col])
                for c in range(_C):
                    o = idx_vec + (c * _V) if c else idx_vec
                    out_v[b, c, pl.ds(e0, _L)] = plsc.load_gather(tab_v, [o])

        issue_in(0, 0)
        issue_in(1, 1)

        for b in range(2):
            wait_in(b)
            compute(b)
            issue_in(2 + b, b)
            issue_out(b, b)

        def pair(t, carry):
            s = t * 2
            for b in range(2):
                wait_in(b)
                wait_out(b)
                compute(b)
                issue_in(s + 2 + b, b)
                issue_out(s + b, b)
            return carry

        lax.fori_loop(1, steps // 2, pair, 0)

        for b in range(2):
            wait_in(b)
            wait_out(b)

    return k


def kernel(primary, table):
    n_rows, n_cols = primary.shape
    tab_t = jnp.zeros((128,), jnp.float32).at[: _C * _V].set(
        table.T.reshape(-1))
    out = _sc_lookup(n_rows, n_cols)(primary.astype(jnp.int32), tab_t)
    return out.reshape(1, _C, n_rows * n_cols)
